# Initial kernel scaffold; baseline (speedup 1.0000x reference)
import functools

import jax
import jax.numpy as jnp
from jax import lax
from jax.experimental import pallas as pl
from jax.experimental.pallas import tpu as pltpu
from jax.experimental.pallas import tpu_sc as plsc

BATCH = 16384
HIST = 200
HIDDEN = 64
NUM_ROWS = 1000

NC, NS = 2, 16
NW = NC * NS
SUB = 128
B = BATCH * HIST
NB = B // SUB
SUBS_PER_W = NB // NW
NSUB = 4
NCHUNK = SUBS_PER_W // NSUB

_mesh = plsc.VectorSubcoreMesh(core_axis_name="c", subcore_axis_name="s")


@functools.partial(
    pl.kernel,
    out_type=jax.ShapeDtypeStruct((NB, SUB, HIDDEN), jnp.float32),
    mesh=_mesh,
    scratch_types=[
        pltpu.VMEM((NSUB, SUB), jnp.int32),
        pltpu.VMEM((NSUB, SUB, HIDDEN), jnp.float32),
        pltpu.SemaphoreType.DMA,
    ],
)
def _emb_lookup(idx_hbm, table_hbm, out_hbm, idx_v, rows_v, gsem):
  wid = lax.axis_index("s") * NC + lax.axis_index("c")
  wbase = wid * SUBS_PER_W

  def body(i, carry):
    row = wbase + i * NSUB
    pltpu.sync_copy(idx_hbm.at[pl.ds(row, NSUB)], idx_v)
    copies = [
        pltpu.async_copy(table_hbm.at[idx_v.at[j]], rows_v.at[j], gsem)
        for j in range(NSUB)
    ]
    for cp in copies:
      cp.wait()
    pltpu.sync_copy(rows_v, out_hbm.at[pl.ds(row, NSUB)])
    return carry

  lax.fori_loop(0, NCHUNK, body, 0)


def kernel(week_numbers, week_embed):
  idx = week_numbers.astype(jnp.int32).reshape(NB, SUB)
  out = _emb_lookup(idx, week_embed)
  return out.reshape(BATCH, HIST, HIDDEN)


# speedup vs baseline: 4.1555x; 4.1555x over previous
import functools

import jax
import jax.numpy as jnp
from jax import lax
from jax.experimental import pallas as pl
from jax.experimental.pallas import tpu as pltpu
from jax.experimental.pallas import tpu_sc as plsc

BATCH = 16384
HIST = 200
HIDDEN = 64
NUM_ROWS = 1000

NC, NS = 2, 16
NW = NC * NS
SUB = 128
B = BATCH * HIST
NB = B // SUB
SUBS_PER_W = NB // NW
NSUB = 4
NCHUNK = SUBS_PER_W // NSUB

_mesh = plsc.VectorSubcoreMesh(core_axis_name="c", subcore_axis_name="s")


@functools.partial(
    pl.kernel,
    out_type=jax.ShapeDtypeStruct((NB, SUB, HIDDEN), jnp.float32),
    mesh=_mesh,
    scratch_types=[
        pltpu.VMEM((NSUB, SUB), jnp.int32),
        pltpu.VMEM((NSUB, SUB, HIDDEN), jnp.float32),
        pltpu.SemaphoreType.DMA,
    ],
    compiler_params=pltpu.CompilerParams(use_tc_tiling_on_sc=False),
)
def _emb_lookup(idx_hbm, table_hbm, out_hbm, idx_v, rows_v, gsem):
  wid = lax.axis_index("s") * NC + lax.axis_index("c")
  wbase = wid * SUBS_PER_W

  def body(i, carry):
    row = wbase + i * NSUB
    pltpu.sync_copy(idx_hbm.at[pl.ds(row, NSUB)], idx_v)
    copies = [
        pltpu.async_copy(table_hbm.at[idx_v.at[j]], rows_v.at[j], gsem)
        for j in range(NSUB)
    ]
    for cp in copies:
      cp.wait()
    pltpu.sync_copy(rows_v, out_hbm.at[pl.ds(row, NSUB)])
    return carry

  lax.fori_loop(0, NCHUNK, body, 0)


def kernel(week_numbers, week_embed):
  idx = week_numbers.astype(jnp.int32).reshape(NB, SUB)
  out = _emb_lookup(idx, week_embed)
  return out.reshape(BATCH, HIST, HIDDEN)


# speedup vs baseline: 4.1605x; 1.0012x over previous
import functools

import jax
import jax.numpy as jnp
from jax import lax
from jax.experimental import pallas as pl
from jax.experimental.pallas import tpu as pltpu
from jax.experimental.pallas import tpu_sc as plsc

BATCH = 16384
HIST = 200
HIDDEN = 64

NC, NS = 2, 16
NW = NC * NS
SUB = 128
B = BATCH * HIST
NB = B // SUB
SUBS_PER_W = NB // NW
NSUB = 4
NCHUNK = SUBS_PER_W // NSUB

_mesh = plsc.VectorSubcoreMesh(core_axis_name="c", subcore_axis_name="s")


@functools.partial(
    pl.kernel,
    out_type=jax.ShapeDtypeStruct((NB, SUB, HIDDEN), jnp.float32),
    mesh=_mesh,
    scratch_types=[
        pltpu.VMEM((2, NSUB, SUB), jnp.int32),
        pltpu.VMEM((2, NSUB, SUB, HIDDEN), jnp.float32),
        pltpu.SemaphoreType.DMA,
        pltpu.SemaphoreType.DMA,
        pltpu.SemaphoreType.DMA,
        pltpu.SemaphoreType.DMA,
        pltpu.SemaphoreType.DMA,
        pltpu.SemaphoreType.DMA,
    ],
    compiler_params=pltpu.CompilerParams(use_tc_tiling_on_sc=False),
)
def _emb_lookup(idx_hbm, table_hbm, out_hbm, idx_v, rows_v,
                is0, is1, gs0, gs1, os0, os1):
  isems = (is0, is1)
  gsems = (gs0, gs1)
  osems = (os0, os1)
  wid = lax.axis_index("s") * NC + lax.axis_index("c")
  wbase = wid * SUBS_PER_W

  def idx_copy(i, b):
    return pltpu.make_async_copy(
        idx_hbm.at[pl.ds(wbase + i * NSUB, NSUB)], idx_v.at[b], isems[b])

  def gather_copies(b):
    return [
        pltpu.make_async_copy(
            table_hbm.at[idx_v.at[b, j]], rows_v.at[b, j], gsems[b])
        for j in range(NSUB)
    ]

  def out_copy(i, b):
    return pltpu.make_async_copy(
        rows_v.at[b], out_hbm.at[pl.ds(wbase + i * NSUB, NSUB)], osems[b])

  idx_copy(0, 0).start()

  def body(g, carry):
    for b in range(2):
      i = 2 * g + b
      idx_copy(i, b).wait()

      @pl.when(i >= 2)
      def _():
        out_copy(i - 2, b).wait()

      for cp in gather_copies(b):
        cp.start()

      @pl.when(i + 1 < NCHUNK)
      def _():
        idx_copy(i + 1, 1 - b).start()

      for cp in gather_copies(b):
        cp.wait()
      out_copy(i, b).start()
    return carry

  lax.fori_loop(0, NCHUNK // 2, body, 0)
  out_copy(NCHUNK - 2, 0).wait()
  out_copy(NCHUNK - 1, 1).wait()


def kernel(week_numbers, week_embed):
  idx = week_numbers.astype(jnp.int32).reshape(NB, SUB)
  out = _emb_lookup(idx, week_embed)
  return out.reshape(BATCH, HIST, HIDDEN)
